# baseline (speedup 1.0000x reference)
import jax
import jax.numpy as jnp
from jax import lax
from jax.experimental import pallas as pl
from jax.experimental.pallas import tpu as pltpu
from jax.experimental.pallas import tpu_sc as plsc

H = 512
W = 512
C = 96
CP = 128
HW = H * W

NC = 2
NS = 16
NW = NC * NS
PPW = HW // NW
P = 64
NCHUNK = PPW // P
L = 16
CB = C // L


def _bcast_lane(v, j):
    idx = jnp.full((L,), j, dtype=jnp.int32)
    return lax.gather(
        v, idx[:, None],
        lax.GatherDimensionNumbers(
            offset_dims=(), collapsed_slice_dims=(0,), start_index_map=(0,)),
        slice_sizes=(1,),
        mode=lax.GatherScatterMode.PROMISE_IN_BOUNDS)


def _sc_grid_sample(tab, gxy, pad):
    mesh = plsc.VectorSubcoreMesh(core_axis_name="c", subcore_axis_name="s")

    def body(tab_hbm, gxy_hbm, pad_hbm, out_hbm,
             gxyv, padv, idxs, ws, rs, outs, gsems, osems):
        wid = lax.axis_index("s") * NC + lax.axis_index("c")
        base = wid * PPW
        pltpu.sync_copy(gxy_hbm.at[pl.ds(base * 2, PPW * 2)], gxyv)
        pltpu.sync_copy(pad_hbm.at[pl.ds(base, PPW)], padv)
        lane2 = lax.iota(jnp.int32, L) * 2

        def fire(ci, s):
            idxv = idxs[s]
            wv = ws[s]
            for g in range(P // L):
                o = g * L
                gbase = ci * (2 * P) + 2 * o
                gx16 = plsc.load_gather(gxyv, [lane2 + gbase])
                gy16 = plsc.load_gather(gxyv, [lane2 + (gbase + 1)])
                pad16 = padv[pl.ds(ci * P + o, L)]
                ix = ((gx16 + 1.0) * W - 1.0) / 2.0
                iy = ((gy16 + 1.0) * H - 1.0) / 2.0
                tx = ix.astype(jnp.int32)
                ty = iy.astype(jnp.int32)
                x0 = jnp.where(ix < tx.astype(jnp.float32), tx - 1, tx)
                y0 = jnp.where(iy < ty.astype(jnp.float32), ty - 1, ty)
                wx1 = ix - x0.astype(jnp.float32)
                wy1 = iy - y0.astype(jnp.float32)
                wx0 = 1.0 - wx1
                wy0 = 1.0 - wy1
                x1 = x0 + 1
                y1 = y0 + 1
                vx0 = (x0 >= 0) & (x0 <= W - 1)
                vx1 = (x1 >= 0) & (x1 <= W - 1)
                vy0 = (y0 >= 0) & (y0 <= H - 1)
                vy1 = (y1 >= 0) & (y1 <= H - 1)
                zero = jnp.zeros((L,), jnp.float32)
                w00 = jnp.where(vy0 & vx0, wy0 * wx0, zero)
                w01 = jnp.where(vy0 & vx1, wy0 * wx1, zero)
                w10 = jnp.where(vy1 & vx0, wy1 * wx0, zero)
                w11 = jnp.where(vy1 & vx1, wy1 * wx1, zero)
                m = w00 + w01 + w10 + w11
                x0c = jnp.clip(x0, 0, W - 1)
                x1c = jnp.clip(x1, 0, W - 1)
                yb0 = jnp.clip(y0, 0, H - 1) * W
                yb1 = jnp.clip(y1, 0, H - 1) * W
                idxv[0, pl.ds(o, L)] = yb0 + x0c
                idxv[1, pl.ds(o, L)] = yb0 + x1c
                idxv[2, pl.ds(o, L)] = yb1 + x0c
                idxv[3, pl.ds(o, L)] = yb1 + x1c
                wv[0, pl.ds(o, L)] = w00 * m
                wv[1, pl.ds(o, L)] = w01 * m
                wv[2, pl.ds(o, L)] = w10 * m
                wv[3, pl.ds(o, L)] = w11 * m
                wv[4, pl.ds(o, L)] = pad16 * (1.0 - m)
            for k in range(4):
                pltpu.async_copy(tab_hbm.at[idxv.at[k]], rs[s][k], gsems[s])

        def drain_combine(ci, s):
            off = base + ci * P
            for k in range(4):
                pltpu.make_async_copy(
                    tab_hbm.at[idxs[s].at[k]], rs[s][k], gsems[s]).wait()
            @pl.when(ci >= 2)
            def _():
                pltpu.make_async_copy(
                    outs[s], out_hbm.at[pl.ds(off - 2 * P, P)], osems[s]).wait()
            r0, r1, r2, r3 = rs[s]
            wv = ws[s]
            outv = outs[s]

            def comb(g, _):
                o = g * L
                w00g = wv[0, pl.ds(o, L)]
                w01g = wv[1, pl.ds(o, L)]
                w10g = wv[2, pl.ds(o, L)]
                w11g = wv[3, pl.ds(o, L)]
                ptg = wv[4, pl.ds(o, L)]
                for j in range(L):
                    p = o + j
                    b00 = _bcast_lane(w00g, j)
                    b01 = _bcast_lane(w01g, j)
                    b10 = _bcast_lane(w10g, j)
                    b11 = _bcast_lane(w11g, j)
                    bpt = _bcast_lane(ptg, j)
                    for cb in range(CB):
                        cs = cb * L
                        acc = b00 * r0[p, pl.ds(cs, L)] + bpt
                        acc = acc + b01 * r1[p, pl.ds(cs, L)]
                        acc = acc + b10 * r2[p, pl.ds(cs, L)]
                        acc = acc + b11 * r3[p, pl.ds(cs, L)]
                        outv[p, pl.ds(cs, L)] = acc
                return 0

            lax.fori_loop(0, P // L, comb, 0)
            pltpu.async_copy(outv, out_hbm.at[pl.ds(off, P)], osems[s])

        fire(0, 0)

        def body2(k2, _):
            ci = k2 * 2

            @pl.when(ci + 1 < NCHUNK)
            def _():
                fire(ci + 1, 1)

            drain_combine(ci, 0)

            @pl.when(ci + 2 < NCHUNK)
            def _():
                fire(ci + 2, 0)

            @pl.when(ci + 1 < NCHUNK)
            def _():
                drain_combine(ci + 1, 1)

            return 0

        lax.fori_loop(0, (NCHUNK + 1) // 2, body2, 0)

        pltpu.make_async_copy(
            outs[0], out_hbm.at[pl.ds(base + (NCHUNK - 2) * P, P)], osems[0]).wait()
        pltpu.make_async_copy(
            outs[1], out_hbm.at[pl.ds(base + (NCHUNK - 1) * P, P)], osems[1]).wait()

    f = pl.kernel(
        body,
        out_type=jax.ShapeDtypeStruct((HW, CP), jnp.float32),
        mesh=mesh,
        scratch_types=[
            pltpu.VMEM((PPW * 2,), jnp.float32),
            pltpu.VMEM((PPW,), jnp.float32),
            [pltpu.VMEM((4, P), jnp.int32) for _ in range(2)],
            [pltpu.VMEM((5, P), jnp.float32) for _ in range(2)],
            [[pltpu.VMEM((P, CP), jnp.float32) for _ in range(4)]
             for _ in range(2)],
            [pltpu.VMEM((P, CP), jnp.float32) for _ in range(2)],
            [pltpu.SemaphoreType.DMA for _ in range(2)],
            [pltpu.SemaphoreType.DMA for _ in range(2)],
        ],
        compiler_params=pltpu.CompilerParams(needs_layout_passes=False),
    )
    return f(tab, gxy, pad)


_TT = 2048


def _tc_pad_transpose(x):
    def body(x_ref, o_ref):
        o_ref[:, :C] = x_ref[...].T
        o_ref[:, C:] = jnp.zeros((_TT, CP - C), jnp.float32)

    return pl.pallas_call(
        body,
        grid=(HW // _TT,),
        in_specs=[pl.BlockSpec((C, _TT), lambda i: (0, i))],
        out_specs=pl.BlockSpec((_TT, CP), lambda i: (i, 0)),
        out_shape=jax.ShapeDtypeStruct((HW, CP), jnp.float32),
        name="tc_pad_transpose",
    )(x)


def _tc_slice_transpose(y):
    def body(y_ref, o_ref):
        o_ref[...] = y_ref[:, :C].T

    return pl.pallas_call(
        body,
        grid=(HW // _TT,),
        in_specs=[pl.BlockSpec((_TT, CP), lambda i: (i, 0))],
        out_specs=pl.BlockSpec((C, _TT), lambda i: (0, i)),
        out_shape=jax.ShapeDtypeStruct((C, HW), jnp.float32),
        name="tc_slice_transpose",
    )(y)


def kernel(input, grid, input_mask, padding_buf):
    tab = _tc_pad_transpose(input[0].reshape(C, HW))
    gxy = grid.reshape(HW * 2)
    pad = padding_buf.reshape(HW)
    out_t = _sc_grid_sample(tab, gxy, pad)
    return _tc_slice_transpose(out_t).reshape(1, C, H, W)
